# baseline (speedup 1.0000x reference)
import jax
import jax.numpy as jnp
from jax import lax
from jax.experimental import pallas as pl
from jax.experimental.pallas import tpu as pltpu
from jax.experimental.pallas import tpu_sc as plsc

NC = 2
NS = 16
NW = NC * NS
B = 4096
D = 64
PLEN = 20
PPAD = 32
BPW = B // NW
VP = 500000


def _sc_gather_body(idx_hbm, w0p_hbm, out_hbm, idx_v, rows_v, sem):
    wid = lax.axis_index("s") * NC + lax.axis_index("c")
    base = wid * BPW
    pltpu.sync_copy(idx_hbm.at[pl.ds(base, BPW)], idx_v)
    pltpu.async_copy(w0p_hbm.at[idx_v], rows_v, sem).wait()
    pltpu.sync_copy(rows_v, out_hbm.at[pl.ds(base, BPW)])


def _sc_gather(pair_idx, W0p):
    mesh = plsc.VectorSubcoreMesh(core_axis_name="c", subcore_axis_name="s",
                                  num_cores=NC, num_subcores=NS)
    return pl.kernel(
        _sc_gather_body,
        out_type=jax.ShapeDtypeStruct((B, 2 * D), jnp.float32),
        mesh=mesh,
        scratch_types=[
            pltpu.VMEM((BPW,), jnp.int32),
            pltpu.VMEM((BPW, 2 * D), jnp.float32),
            pltpu.SemaphoreType.DMA,
        ],
    )(pair_idx, W0p)


def _tc_body(p0i_ref, c2_ref, par_ref, labels_ref, w1_ref, out_ref,
             p0b, psem):
    for j in range(PLEN):
        pltpu.make_async_copy(w1_ref.at[pl.ds(p0i_ref[j], 1)],
                              p0b.at[pl.ds(j, 1)], psem).start()
    pltpu.make_async_copy(w1_ref.at[pl.ds(0, PLEN)],
                          p0b.at[pl.ds(0, PLEN)], psem).wait()

    c2 = c2_ref[...]
    par = par_ref[...]
    c = jnp.where(par > 0.5, c2[:, D:], c2[:, :D])
    p = p0b[...]
    scores = lax.dot_general(c, p, (((1,), (1,)), ((), ())),
                             preferred_element_type=jnp.float32)
    s = scores[:, :PLEN]
    lab = labels_ref[...]
    z = jnp.log(1.0 / (1.0 + jnp.exp(-s)))
    log_z = jnp.maximum(jnp.log(z), -100.0)
    log_1mz = jnp.maximum(jnp.log(1.0 - z), -100.0)
    out_ref[0, 0] = -jnp.sum(lab * log_z + (1.0 - lab) * log_1mz)


def _tc_loss(paths0, c2, parity, labels, W1):
    out = pl.pallas_call(
        _tc_body,
        out_shape=jax.ShapeDtypeStruct((1, 1), jnp.float32),
        in_specs=[
            pl.BlockSpec(memory_space=pltpu.SMEM),
            pl.BlockSpec(memory_space=pltpu.VMEM),
            pl.BlockSpec(memory_space=pltpu.VMEM),
            pl.BlockSpec(memory_space=pltpu.VMEM),
            pl.BlockSpec(memory_space=pl.ANY),
        ],
        out_specs=pl.BlockSpec(memory_space=pltpu.SMEM),
        scratch_shapes=[
            pltpu.VMEM((PPAD, D), jnp.float32),
            pltpu.SemaphoreType.DMA,
        ],
    )(paths0, c2, parity, labels, W1)
    return out[0, 0]


RPB = 1000


def _repack_body(lo_ref, hi_ref, out_ref):
    out_ref[...] = jnp.concatenate([lo_ref[...], hi_ref[...]], axis=1)


def _repack(W0):
    return pl.pallas_call(
        _repack_body,
        grid=(VP // RPB,),
        in_specs=[
            pl.BlockSpec((RPB, D), lambda i: (i, 0)),
            pl.BlockSpec((RPB, D), lambda i: (VP // RPB + i, 0)),
        ],
        out_specs=pl.BlockSpec((RPB, 2 * D), lambda i: (i, 0)),
        out_shape=jax.ShapeDtypeStruct((VP, 2 * D), jnp.float32),
    )(W0, W0)


def kernel(c_words, paths, labels, W0, W1):
    c_words = jnp.squeeze(c_words).astype(jnp.int32)
    paths0 = jnp.squeeze(paths)[0].astype(jnp.int32)
    labels = jnp.squeeze(labels)
    W0p = _repack(W0)
    pair_idx = jnp.where(c_words >= VP, c_words - VP, c_words)
    parity = (c_words >= VP).astype(jnp.float32).reshape(B, 1)
    c2 = _sc_gather(pair_idx, W0p)
    return _tc_loss(paths0, c2, parity, labels, W1)


# speedup vs baseline: 3.8380x; 3.8380x over previous
import jax
import jax.numpy as jnp
from jax import lax
from jax.experimental import pallas as pl
from jax.experimental.pallas import tpu as pltpu

V = 999999
B = 4096
D = 64
PLEN = 20
PPAD = 32
CH = 8192
NQ = 8


def _p0_body(p0i_ref, w1t_ref, out_ref, slab, psem):
    for j in range(PLEN):
        base = (p0i_ref[j] // 128) * 128
        pltpu.make_async_copy(w1t_ref.at[:, pl.ds(base, 128)],
                              slab.at[j], psem).start()
    for j in range(PLEN):
        pltpu.make_async_copy(w1t_ref.at[:, pl.ds(0, 128)],
                              slab.at[j], psem).wait()

    sl = slab[...]
    i0 = lax.broadcasted_iota(jnp.int32, (PLEN, 1, 128), 0)
    i2 = lax.broadcasted_iota(jnp.int32, (PLEN, 1, 128), 2)
    rem = jnp.zeros((PLEN, 1, 128), jnp.int32)
    for j in range(PLEN):
        rem = jnp.where(i0 == j, p0i_ref[j] % 128, rem)
    oh = (i2 == rem).astype(jnp.float32)
    sel = jnp.sum(sl * oh, axis=2)
    out_ref[...] = jnp.concatenate(
        [sel, jnp.zeros((PPAD - PLEN, D), jnp.float32)], axis=0)


def _p0T(paths0, W1T):
    return pl.pallas_call(
        _p0_body,
        out_shape=jax.ShapeDtypeStruct((PPAD, D), jnp.float32),
        in_specs=[
            pl.BlockSpec(memory_space=pltpu.SMEM),
            pl.BlockSpec(memory_space=pl.ANY),
        ],
        out_specs=pl.BlockSpec(memory_space=pltpu.VMEM),
        scratch_shapes=[
            pltpu.VMEM((PLEN, D, 128), jnp.float32),
            pltpu.SemaphoreType.DMA,
        ],
    )(paths0, W1T)


def _sweep_body(w0t_ref, p0t_ref, yt_ref):
    yt_ref[...] = lax.dot_general(w0t_ref[...], p0t_ref[...],
                                  (((0,), (1,)), ((), ())),
                                  preferred_element_type=jnp.float32)


def _sweep(W0T, p0t):
    n = (V + CH - 1) // CH
    return pl.pallas_call(
        _sweep_body,
        grid=(n,),
        in_specs=[
            pl.BlockSpec((D, CH), lambda i: (0, i)),
            pl.BlockSpec((PPAD, D), lambda i: (0, 0)),
        ],
        out_specs=pl.BlockSpec((CH, PPAD), lambda i: (i, 0)),
        out_shape=jax.ShapeDtypeStruct((V, PPAD), jnp.float32),
    )(W0T, p0t)


def _gather_body(cw_ref, yt_ref, labels_ref, out_ref, rows, sem):
    def issue(step, _):
        for j in range(NQ):
            b = step * NQ + j
            pltpu.make_async_copy(yt_ref.at[pl.ds(cw_ref[b], 1)],
                                  rows.at[pl.ds(b, 1)], sem.at[j]).start()
        return 0

    lax.fori_loop(0, B // NQ, issue, 0)
    for j in range(NQ):
        pltpu.make_async_copy(yt_ref.at[pl.ds(0, B // NQ)],
                              rows.at[pl.ds(0, B // NQ)], sem.at[j]).wait()

    s = rows[...][:, :PLEN]
    lab = labels_ref[...]
    z = jnp.log(1.0 / (1.0 + jnp.exp(-s)))
    log_z = jnp.maximum(jnp.log(z), -100.0)
    log_1mz = jnp.maximum(jnp.log(1.0 - z), -100.0)
    out_ref[0, 0] = -jnp.sum(lab * log_z + (1.0 - lab) * log_1mz)


def _gather_loss(c_words, yt, labels):
    out = pl.pallas_call(
        _gather_body,
        out_shape=jax.ShapeDtypeStruct((1, 1), jnp.float32),
        in_specs=[
            pl.BlockSpec(memory_space=pltpu.SMEM),
            pl.BlockSpec(memory_space=pl.ANY),
            pl.BlockSpec(memory_space=pltpu.VMEM),
        ],
        out_specs=pl.BlockSpec(memory_space=pltpu.SMEM),
        scratch_shapes=[
            pltpu.VMEM((B, PPAD), jnp.float32),
            pltpu.SemaphoreType.DMA((NQ,)),
        ],
    )(c_words, yt, labels)
    return out[0, 0]


def kernel(c_words, paths, labels, W0, W1):
    c_words = jnp.squeeze(c_words).astype(jnp.int32)
    paths0 = jnp.squeeze(paths)[0].astype(jnp.int32)
    labels = jnp.squeeze(labels)
    W0T = jnp.swapaxes(W0, 0, 1)
    W1T = jnp.swapaxes(W1, 0, 1)
    p0t = _p0T(paths0, W1T)
    yt = _sweep(W0T, p0t)
    return _gather_loss(c_words, yt, labels)


# speedup vs baseline: 4.3057x; 1.1219x over previous
import jax
import jax.numpy as jnp
from jax import lax
from jax.experimental import pallas as pl
from jax.experimental.pallas import tpu as pltpu

V = 999999
B = 4096
D = 64
PLEN = 20
PPAD = 32
CH = 32768
NQ = 8


def _p0_body(p0i_ref, w1t_ref, out_ref, slab, psem):
    for j in range(PLEN):
        base = (p0i_ref[j] // 128) * 128
        pltpu.make_async_copy(w1t_ref.at[:, pl.ds(base, 128)],
                              slab.at[j], psem).start()
    for j in range(PLEN):
        pltpu.make_async_copy(w1t_ref.at[:, pl.ds(0, 128)],
                              slab.at[j], psem).wait()

    sl = slab[...]
    i0 = lax.broadcasted_iota(jnp.int32, (PLEN, 1, 128), 0)
    i2 = lax.broadcasted_iota(jnp.int32, (PLEN, 1, 128), 2)
    rem = jnp.zeros((PLEN, 1, 128), jnp.int32)
    for j in range(PLEN):
        rem = jnp.where(i0 == j, p0i_ref[j] % 128, rem)
    oh = (i2 == rem).astype(jnp.float32)
    sel = jnp.sum(sl * oh, axis=2)
    out_ref[...] = jnp.concatenate(
        [sel, jnp.zeros((PPAD - PLEN, D), jnp.float32)], axis=0)


def _p0T(paths0, W1T):
    return pl.pallas_call(
        _p0_body,
        out_shape=jax.ShapeDtypeStruct((PPAD, D), jnp.float32),
        in_specs=[
            pl.BlockSpec(memory_space=pltpu.SMEM),
            pl.BlockSpec(memory_space=pl.ANY),
        ],
        out_specs=pl.BlockSpec(memory_space=pltpu.VMEM),
        scratch_shapes=[
            pltpu.VMEM((PLEN, D, 128), jnp.float32),
            pltpu.SemaphoreType.DMA,
        ],
    )(paths0, W1T)


def _sweep_body(w0t_ref, p0t_ref, yt_ref):
    yt_ref[...] = lax.dot_general(w0t_ref[...], p0t_ref[...],
                                  (((0,), (1,)), ((), ())),
                                  preferred_element_type=jnp.float32)


def _sweep(W0T, p0t):
    n = (V + CH - 1) // CH
    return pl.pallas_call(
        _sweep_body,
        grid=(n,),
        in_specs=[
            pl.BlockSpec((D, CH), lambda i: (0, i)),
            pl.BlockSpec((PPAD, D), lambda i: (0, 0)),
        ],
        out_specs=pl.BlockSpec((CH, PPAD), lambda i: (i, 0)),
        out_shape=jax.ShapeDtypeStruct((V, PPAD), jnp.float32),
    )(W0T, p0t)


def _gather_body(cw_ref, yt_ref, labels_ref, out_ref, rows, sem):
    def issue(step, _):
        for j in range(NQ):
            b = step * NQ + j
            pltpu.make_async_copy(yt_ref.at[pl.ds(cw_ref[b], 1)],
                                  rows.at[pl.ds(b, 1)], sem.at[j]).start()
        return 0

    lax.fori_loop(0, B // NQ, issue, 0)
    for j in range(NQ):
        pltpu.make_async_copy(yt_ref.at[pl.ds(0, B // NQ)],
                              rows.at[pl.ds(0, B // NQ)], sem.at[j]).wait()

    s = rows[...][:, :PLEN]
    lab = labels_ref[...]
    z = jnp.log(1.0 / (1.0 + jnp.exp(-s)))
    log_z = jnp.maximum(jnp.log(z), -100.0)
    log_1mz = jnp.maximum(jnp.log(1.0 - z), -100.0)
    out_ref[0, 0] = -jnp.sum(lab * log_z + (1.0 - lab) * log_1mz)


def _gather_loss(c_words, yt, labels):
    out = pl.pallas_call(
        _gather_body,
        out_shape=jax.ShapeDtypeStruct((1, 1), jnp.float32),
        in_specs=[
            pl.BlockSpec(memory_space=pltpu.SMEM),
            pl.BlockSpec(memory_space=pl.ANY),
            pl.BlockSpec(memory_space=pltpu.VMEM),
        ],
        out_specs=pl.BlockSpec(memory_space=pltpu.SMEM),
        scratch_shapes=[
            pltpu.VMEM((B, PPAD), jnp.float32),
            pltpu.SemaphoreType.DMA((NQ,)),
        ],
    )(c_words, yt, labels)
    return out[0, 0]


def kernel(c_words, paths, labels, W0, W1):
    c_words = jnp.squeeze(c_words).astype(jnp.int32)
    paths0 = jnp.squeeze(paths)[0].astype(jnp.int32)
    labels = jnp.squeeze(labels)
    W0T = jnp.swapaxes(W0, 0, 1)
    W1T = jnp.swapaxes(W1, 0, 1)
    p0t = _p0T(paths0, W1T)
    yt = _sweep(W0T, p0t)
    return _gather_loss(c_words, yt, labels)
